# baseline (speedup 1.0000x reference)
import functools
import math

import jax
import jax.numpy as jnp
from jax import lax
from jax.experimental import pallas as pl
from jax.experimental.pallas import tpu as pltpu
from jax.experimental.pallas import tpu_sc as plsc

_SCALE = math.sqrt(128.0)
_CHUNK = 128
_NBUF = 4


def _make_gather(vocab, dim, n_idx):
    info = plsc.get_sparse_core_info()
    nc, ns = info.num_cores, info.num_subcores
    nw = nc * ns
    assert n_idx % (nw * _CHUNK) == 0
    per_w = n_idx // nw
    n_chunks = per_w // _CHUNK
    assert n_chunks % _NBUF == 0
    n_groups = n_chunks // _NBUF

    mesh = plsc.VectorSubcoreMesh(core_axis_name="c", subcore_axis_name="s")

    @functools.partial(
        pl.kernel,
        mesh=mesh,
        compiler_params=pltpu.CompilerParams(use_tc_tiling_on_sc=False),
        out_type=jax.ShapeDtypeStruct((n_idx, dim), jnp.float32),
        scratch_types=[
            pltpu.VMEM((n_chunks, _CHUNK), jnp.int32),
            *([pltpu.VMEM((_CHUNK, dim // 2), jnp.int32)] * _NBUF),
            *([pltpu.VMEM((_CHUNK, dim), jnp.float32)] * _NBUF),
            *([pltpu.SemaphoreType.DMA] * (2 * _NBUF)),
        ],
    )
    def gather_k(table_hbm, idx_hbm, out_hbm, idx_v, *bufs_and_sems):
        rows = bufs_and_sems[:_NBUF]
        obuf = bufs_and_sems[_NBUF : 2 * _NBUF]
        gsem = bufs_and_sems[2 * _NBUF : 3 * _NBUF]
        osem = bufs_and_sems[3 * _NBUF :]
        wid = lax.axis_index("s") * nc + lax.axis_index("c")
        base = wid * per_w
        pltpu.sync_copy(idx_hbm.at[pl.ds(wid * n_chunks, n_chunks)], idx_v)

        def body(g, carry):
            first = g * _NBUF
            for b in range(_NBUF):
                @pl.when(g > 0)
                def _():
                    pltpu.make_async_copy(
                        obuf[b], out_hbm.at[pl.ds(0, _CHUNK)], osem[b]
                    ).wait()
                pltpu.async_copy(
                    table_hbm.at[idx_v.at[first + b]], rows[b], gsem[b]
                )
            for b in range(_NBUF):
                pltpu.make_async_copy(
                    table_hbm.at[idx_v.at[first + b]], rows[b], gsem[b]
                ).wait()

                def wbody(g2, c, src=rows[b], dst=obuf[b]):
                    for rr in range(4):
                        r = g2 * 4 + rr
                        for j in range(dim // 32):
                            v = src[r, pl.ds(j * 16, 16)]
                            lo = lax.bitcast_convert_type(v << 16, jnp.float32)
                            hi = lax.bitcast_convert_type(
                                (v >> 16) << 16, jnp.float32
                            )
                            dst[r, pl.ds(j * 16, 16)] = lo * _SCALE
                            dst[r, pl.ds(dim // 2 + j * 16, 16)] = hi * _SCALE
                    return c

                lax.fori_loop(0, _CHUNK // 4, wbody, 0)
                off = base + (first + b) * _CHUNK
                pltpu.async_copy(obuf[b], out_hbm.at[pl.ds(off, _CHUNK)], osem[b])
            return carry

        lax.fori_loop(0, n_groups, body, 0)
        for b in range(_NBUF):
            pltpu.make_async_copy(
                obuf[b], out_hbm.at[pl.ds(0, _CHUNK)], osem[b]
            ).wait()

    return gather_k


def kernel(x, table):
    vocab, dim = table.shape
    x_flat = x.reshape(-1).astype(jnp.int32)
    n_idx = x_flat.shape[0]
    bits = lax.bitcast_convert_type(table.astype(jnp.bfloat16), jnp.uint16)
    a = bits[:, : dim // 2].astype(jnp.uint32)
    bhalf = bits[:, dim // 2 :].astype(jnp.uint32)
    packed = lax.bitcast_convert_type(a | (bhalf << 16), jnp.int32)
    idx2d = x_flat.reshape(-1, _CHUNK)
    out = _make_gather(vocab, dim, n_idx)(packed, idx2d)
    return out.reshape(x.shape + (dim,))


# speedup vs baseline: 2.4364x; 2.4364x over previous
import functools
import math

import jax
import jax.numpy as jnp
from jax import lax
from jax.experimental import pallas as pl
from jax.experimental.pallas import tpu as pltpu
from jax.experimental.pallas import tpu_sc as plsc

_SCALE = math.sqrt(128.0)
_CHUNK = 128
_NPAIR = 2


def _make_gather(vocab, dim, n_idx):
    info = plsc.get_sparse_core_info()
    nc, ns = info.num_cores, info.num_subcores
    nw = nc * ns
    assert n_idx % (nw * _CHUNK) == 0
    per_w = n_idx // nw
    n_chunks = per_w // _CHUNK
    assert n_chunks % (2 * _NPAIR) == 0
    n_groups = n_chunks // (2 * _NPAIR)

    mesh = plsc.VectorSubcoreMesh(core_axis_name="c", subcore_axis_name="s")

    @functools.partial(
        pl.kernel,
        mesh=mesh,
        out_type=jax.ShapeDtypeStruct((n_idx, dim), jnp.float32),
        scratch_types=[
            pltpu.VMEM((n_chunks, _CHUNK), jnp.int32),
            *([pltpu.VMEM((2 * _CHUNK, dim), jnp.float32)] * _NPAIR),
            *([pltpu.SemaphoreType.DMA] * (3 * _NPAIR)),
        ],
    )
    def gather_k(table_hbm, idx_hbm, out_hbm, idx_v, *bufs_and_sems):
        rows = bufs_and_sems[:_NPAIR]
        gsem = bufs_and_sems[_NPAIR : 3 * _NPAIR]
        osem = bufs_and_sems[3 * _NPAIR :]
        wid = lax.axis_index("s") * nc + lax.axis_index("c")
        base = wid * per_w
        pltpu.sync_copy(idx_hbm.at[pl.ds(wid * n_chunks, n_chunks)], idx_v)

        def body(g, carry):
            first = g * 2 * _NPAIR
            for p in range(_NPAIR):
                @pl.when(g > 0)
                def _():
                    pltpu.make_async_copy(
                        rows[p], out_hbm.at[pl.ds(0, 2 * _CHUNK)], osem[p]
                    ).wait()
                for h in range(2):
                    pltpu.async_copy(
                        table_hbm.at[idx_v.at[first + 2 * p + h]],
                        rows[p].at[pl.ds(h * _CHUNK, _CHUNK)],
                        gsem[2 * p + h],
                    )
            for p in range(_NPAIR):
                for h in range(2):
                    pltpu.make_async_copy(
                        table_hbm.at[idx_v.at[first + 2 * p + h]],
                        rows[p].at[pl.ds(h * _CHUNK, _CHUNK)],
                        gsem[2 * p + h],
                    ).wait()

                def sbody(r, c, buf=rows[p]):
                    for j in range(dim // 16):
                        buf[r, pl.ds(j * 16, 16)] = (
                            buf[r, pl.ds(j * 16, 16)] * _SCALE
                        )
                    return c

                lax.fori_loop(0, 2 * _CHUNK, sbody, 0)
                off = base + (first + 2 * p) * _CHUNK
                pltpu.async_copy(
                    rows[p], out_hbm.at[pl.ds(off, 2 * _CHUNK)], osem[p]
                )
            return carry

        lax.fori_loop(0, n_groups, body, 0)
        for p in range(_NPAIR):
            pltpu.make_async_copy(
                rows[p], out_hbm.at[pl.ds(0, 2 * _CHUNK)], osem[p]
            ).wait()

    return gather_k


def kernel(x, table):
    vocab, dim = table.shape
    x_flat = x.reshape(-1).astype(jnp.int32)
    n_idx = x_flat.shape[0]
    idx2d = x_flat.reshape(-1, _CHUNK)
    out = _make_gather(vocab, dim, n_idx)(table, idx2d)
    return out.reshape(x.shape + (dim,))
